# baseline (speedup 1.0000x reference)
import functools

import jax
import jax.numpy as jnp
from jax import lax
from jax.experimental import pallas as pl
from jax.experimental.pallas import tpu as pltpu
from jax.experimental.pallas import tpu_sc as plsc

_TEMP = 10.0
_NEG = -1e30


def _mm_body(fs_ref, ft_ref, w_ref, b_ref, ys_ref, yt_ref):
    w = w_ref[...]
    bias = b_ref[...]
    rows = fs_ref.shape[0]
    dn = (((1,), (0,)), ((), ()))
    neg_col = jnp.full((rows, 1), _NEG, jnp.float32)
    y_s = lax.dot_general(fs_ref[...], w, dn,
                          preferred_element_type=jnp.float32) + bias
    y_t = lax.dot_general(ft_ref[...], w, dn,
                          preferred_element_type=jnp.float32) + bias
    ys_ref[...] = jnp.concatenate([y_s, neg_col], axis=1)
    yt_ref[...] = jnp.concatenate([y_t, neg_col], axis=1)


def _tc_matmul(f_s, f_t, W, b2):
    n, d = f_s.shape
    c = W.shape[1]
    rows = 512
    nb = n // rows
    return pl.pallas_call(
        _mm_body,
        grid=(nb,),
        in_specs=[
            pl.BlockSpec((rows, d), lambda i: (i, 0)),
            pl.BlockSpec((rows, d), lambda i: (i, 0)),
            pl.BlockSpec((d, c), lambda i: (0, 0)),
            pl.BlockSpec((1, c), lambda i: (0, 0)),
        ],
        out_specs=[
            pl.BlockSpec((rows, c + 1), lambda i: (i, 0)),
            pl.BlockSpec((rows, c + 1), lambda i: (i, 0)),
        ],
        out_shape=[
            jax.ShapeDtypeStruct((n, c + 1), jnp.float32),
            jax.ShapeDtypeStruct((n, c + 1), jnp.float32),
        ],
    )(f_s, f_t, W, b2)


def _rsqrt_newton(qv):
    iv = plsc.bitcast(qv, jnp.int32)
    r = plsc.bitcast(jnp.int32(0x5F3759DF) - (iv >> 1), jnp.float32)
    for _ in range(3):
        r = r * (1.5 - 0.5 * qv * r * r)
    return r


def _sc_body(rows_per, ys_hbm, yt_hbm, lab_hbm, out_s, out_t,
             yv_s, yv_t, labv, acc_s, acc_t):
    wid = lax.axis_index("s") * 2 + lax.axis_index("c")
    base = wid * rows_per
    iota = lax.iota(jnp.int32, 16)
    zeros16 = jnp.zeros((16,), jnp.float32)

    pltpu.sync_copy(ys_hbm.at[pl.ds(base * 16, rows_per * 16)], yv_s)
    pltpu.sync_copy(yt_hbm.at[pl.ds(base * 16, rows_per * 16)], yv_t)
    pltpu.sync_copy(lab_hbm.at[pl.ds(base, rows_per)], labv)

    for j in range(16):
        acc_s[pl.ds(j * 16, 16)] = zeros16
        acc_t[pl.ds(j * 16, 16)] = zeros16

    ones16 = jnp.ones((16,), jnp.float32)
    inv_t = 1.0 / _TEMP

    def group(g, carry):
        rowidx = g * 16 + iota

        flatbase = rowidx * 16

        def cols(ref):
            return [plsc.load_gather(ref, [flatbase + l]) for l in range(15)]

        v = cols(yv_s)
        m = v[0]
        for l in range(1, 15):
            m = jnp.maximum(m, v[l])
        e = [jnp.exp((v[l] - m) * inv_t) for l in range(15)]
        q = e[0] * e[0]
        for l in range(1, 15):
            q = q + e[l] * e[l]
        rs = _rsqrt_newton(q)
        lab16 = labv[pl.ds(g * 16, 16)]
        sbase = lab16 * 16
        for l in range(15):
            plsc.addupdate_scatter(acc_s, [sbase + l], e[l] * rs)
        plsc.addupdate_scatter(acc_s, [sbase + 15], ones16)

        v = cols(yv_t)
        m = v[0]
        for l in range(1, 15):
            m = jnp.maximum(m, v[l])
        s1 = jnp.exp(v[0] - m)
        for l in range(1, 15):
            s1 = s1 + jnp.exp(v[l] - m)
        conf = jnp.where(s1 < 1.25, 1.0, 0.0)
        e = [jnp.exp((v[l] - m) * inv_t) for l in range(15)]
        q = e[0] * e[0]
        for l in range(1, 15):
            q = q + e[l] * e[l]
        rs = _rsqrt_newton(q) * conf
        lt = jnp.full((16,), 15, jnp.int32)
        for l in range(14, -1, -1):
            lt = jnp.where(v[l] == m, l, lt)
        tbase = lt * 16
        for l in range(15):
            plsc.addupdate_scatter(acc_t, [tbase + l], e[l] * rs)
        plsc.addupdate_scatter(acc_t, [tbase + 15], conf)
        return carry

    lax.fori_loop(0, rows_per // 16, group, 0)

    pltpu.sync_copy(acc_s, out_s.at[wid])
    pltpu.sync_copy(acc_t, out_t.at[wid])


def _sc_stage(ys, yt, label_s):
    n = ys.shape[0]
    nw = 32
    rows_per = n // nw
    mesh = plsc.VectorSubcoreMesh(core_axis_name="c", subcore_axis_name="s")
    f = pl.kernel(
        functools.partial(_sc_body, rows_per),
        out_type=[
            jax.ShapeDtypeStruct((nw, 256), jnp.float32),
            jax.ShapeDtypeStruct((nw, 256), jnp.float32),
        ],
        mesh=mesh,
        compiler_params=pltpu.CompilerParams(needs_layout_passes=False),
        scratch_types=[
            pltpu.VMEM((rows_per * 16,), jnp.float32),
            pltpu.VMEM((rows_per * 16,), jnp.float32),
            pltpu.VMEM((rows_per,), jnp.int32),
            pltpu.VMEM((256,), jnp.float32),
            pltpu.VMEM((256,), jnp.float32),
        ],
    )
    return f(ys.reshape(-1), yt.reshape(-1), label_s)


def _fin_body(ps_ref, pt_ref, out_ref):
    a = jnp.sum(ps_ref[...], axis=0, keepdims=True)
    bt = jnp.sum(pt_ref[...], axis=0, keepdims=True)
    prod = a * bt
    total = jnp.sum(prod)
    lane = lax.broadcasted_iota(jnp.int32, prod.shape, 1)
    count = jnp.sum(jnp.where(lane % 16 == 15, prod, 0.0))
    loss_sum = 2.0 * count - total
    out_ref[0, 0] = (_TEMP * _TEMP) * loss_sum / jnp.maximum(count, 1.0)


def _tc_finish(ps, pt):
    nw = ps.shape[0]
    return pl.pallas_call(
        _fin_body,
        in_specs=[
            pl.BlockSpec((nw, 256), lambda: (0, 0)),
            pl.BlockSpec((nw, 256), lambda: (0, 0)),
        ],
        out_specs=pl.BlockSpec(memory_space=pltpu.SMEM),
        out_shape=jax.ShapeDtypeStruct((1, 1), jnp.float32),
    )(ps, pt)


def kernel(f_s, f_t, label_s, W, b):
    c = W.shape[1]
    b2 = b.reshape(1, c)
    ys, yt = _tc_matmul(f_s, f_t, W, b2)
    ps, pt = _sc_stage(ys, yt, label_s)
    out = _tc_finish(ps, pt)
    return out.reshape(1)


# speedup vs baseline: 1.3729x; 1.3729x over previous
import jax
import jax.numpy as jnp
from jax import lax
from jax.experimental import pallas as pl
from jax.experimental.pallas import tpu as pltpu

_TEMP = 10.0


def _loss_body(lab_ref, fs_ref, ft_ref, w_ref, b_ref, out_ref, acc_s, acc_t):
    i = pl.program_id(0)
    nb = pl.num_programs(0)
    c = w_ref.shape[1]
    rows = fs_ref.shape[0]

    @pl.when(i == 0)
    def _init():
        acc_s[...] = jnp.zeros_like(acc_s)
        acc_t[...] = jnp.zeros_like(acc_t)

    w = w_ref[...]
    bias = b_ref[...]

    fs = fs_ref[...]
    ft = ft_ref[...]
    dn = (((1,), (0,)), ((), ()))
    y_s = lax.dot_general(fs, w, dn, preferred_element_type=jnp.float32) + bias
    y_t = lax.dot_general(ft, w, dn, preferred_element_type=jnp.float32) + bias

    ones_col = jnp.ones((rows, 1), jnp.float32)

    m_s = jnp.max(y_s, axis=1, keepdims=True)
    e_s = jnp.exp((y_s - m_s) / _TEMP)
    an = e_s / jnp.sqrt(jnp.sum(e_s * e_s, axis=1, keepdims=True))
    an1 = jnp.concatenate([an, ones_col], axis=1)

    lbl = lab_ref[0]
    sub = lax.broadcasted_iota(jnp.int32, (c + 1, rows), 0)
    oh_s = (sub == lbl).astype(jnp.float32)
    acc_s[...] += lax.dot_general(
        oh_s, an1, (((1,), (0,)), ((), ())), preferred_element_type=jnp.float32)

    m_t = jnp.max(y_t, axis=1, keepdims=True)
    s_t1 = jnp.sum(jnp.exp(y_t - m_t), axis=1, keepdims=True)
    conf = (1.0 / s_t1) > 0.8

    e_t = jnp.exp((y_t - m_t) / _TEMP)
    bn = e_t / jnp.sqrt(jnp.sum(e_t * e_t, axis=1, keepdims=True))
    bn1 = jnp.concatenate([bn, ones_col], axis=1)

    lane = lax.broadcasted_iota(jnp.int32, y_t.shape, 1)
    is_max = y_t == m_t
    first = jnp.min(jnp.where(is_max, lane, c), axis=1, keepdims=True)
    lane1 = lax.broadcasted_iota(jnp.int32, (rows, c + 1), 1)
    oh_t = ((lane1 == first) & conf).astype(jnp.float32)
    acc_t[...] += lax.dot_general(
        oh_t, bn1, (((0,), (0,)), ((), ())), preferred_element_type=jnp.float32)

    @pl.when(i == nb - 1)
    def _finish():
        prod = acc_s[...] * acc_t[...]
        total = jnp.sum(prod)
        lane2 = lax.broadcasted_iota(jnp.int32, prod.shape, 1)
        count = jnp.sum(jnp.where(lane2 == c, prod, 0.0))
        loss_sum = 2.0 * count - total
        out_ref[0, 0] = (_TEMP * _TEMP) * loss_sum / jnp.maximum(count, 1.0)


def kernel(f_s, f_t, label_s, W, b):
    n, d = f_s.shape
    c = W.shape[1]
    rows = 512
    nb = n // rows

    lab3 = label_s.reshape(nb, 1, rows)
    b2 = b.reshape(1, c)

    out = pl.pallas_call(
        _loss_body,
        grid=(nb,),
        in_specs=[
            pl.BlockSpec((1, 1, rows), lambda i: (i, 0, 0)),
            pl.BlockSpec((rows, d), lambda i: (i, 0)),
            pl.BlockSpec((rows, d), lambda i: (i, 0)),
            pl.BlockSpec((d, c), lambda i: (0, 0)),
            pl.BlockSpec((1, c), lambda i: (0, 0)),
        ],
        out_specs=pl.BlockSpec(memory_space=pltpu.SMEM),
        out_shape=jax.ShapeDtypeStruct((1, 1), jnp.float32),
        scratch_shapes=[
            pltpu.VMEM((c + 1, c + 1), jnp.float32),
            pltpu.VMEM((c + 1, c + 1), jnp.float32),
        ],
    )(lab3, f_s, f_t, W, b2)
    return out.reshape(1)
